# baseline (speedup 1.0000x reference)
import functools

import jax
import jax.numpy as jnp
from jax import lax
from jax.experimental import pallas as pl
from jax.experimental.pallas import tpu as pltpu
from jax.experimental.pallas import tpu_sc as plsc

_GATHER_WINDOW = 128
_TC_BLOCK_TOKENS = 2048
_COMB_ROWS = 128
_NUM_CHUNKS = 8
_F32_CHUNKS = 2


def _sc_gather_chunk(table, ids2, chunk, Tc):
    H = table.shape[1]
    W = _GATHER_WINDOW
    steps = Tc // W
    off = chunk * steps
    mesh = plsc.VectorSubcoreMesh(core_axis_name="c", subcore_axis_name="s")

    @functools.partial(
        pl.kernel,
        out_type=jax.ShapeDtypeStruct((Tc, H), table.dtype),
        mesh=mesh,
    )
    def gather_kernel(x_hbm, i_hbm, o_hbm):
        def body(i_vmem, o_vmem):
            pltpu.sync_copy(x_hbm.at[i_vmem.at[0]], o_vmem)

        pltpu.emit_pipeline(
            body,
            grid=(steps,),
            in_specs=[
                pl.BlockSpec((1, W), index_map=lambda i: (0, i + off))
            ],
            out_specs=[
                pl.BlockSpec((W, H), index_map=lambda i: (i, 0))
            ],
            core_axis_name=("c", "s"),
            dimension_semantics=(pltpu.PARALLEL,),
        )(i_hbm, o_hbm)

    return gather_kernel(table, ids2)


def _tc_fuse_chunk(acc, gathered_c, age_r, tt_r, comb, pos_emb, gamma2, beta2,
                   chunk, T, ln_eps, packed):
    Tc = gathered_c.shape[0]
    H = 2 * gathered_c.shape[1] if packed else gathered_c.shape[1]
    S = pos_emb.shape[0]
    BT = _TC_BLOCK_TOKENS
    NBc = Tc // BT
    KB = BT // S
    AGES = 110
    base = chunk * NBc

    def body(*refs):
        if acc is None:
            g_ref, age_ref, tt_ref, comb_ref, pos_ref, gam_ref, bet_ref, \
                o_ref = refs
        else:
            _, g_ref, age_ref, tt_ref, comb_ref, pos_ref, gam_ref, bet_ref, \
                o_ref = refs
        if packed:
            g32 = g_ref[...]
            lo_f = lax.bitcast_convert_type(g32 << 16, jnp.float32)
            hi_f = lax.bitcast_convert_type(
                jnp.bitwise_and(g32, jnp.int32(-65536)), jnp.float32)
            g = jnp.concatenate([lo_f, hi_f], axis=1)
        else:
            g = g_ref[...]
        age = age_ref[0, 0, :][:, None]
        tt = tt_ref[0, 0, :][:, None]

        col = lax.broadcasted_iota(jnp.int32, (1, _COMB_ROWS), 1)
        oh = ((age == col).astype(jnp.bfloat16)
              + (tt + AGES == col).astype(jnp.bfloat16))
        small_v = jnp.dot(oh, comb_ref[...],
                          preferred_element_type=jnp.float32)

        pos = jnp.broadcast_to(pos_ref[...][None], (KB, S, H)).reshape(BT, H)

        emb = g + small_v + pos
        mean = jnp.mean(emb, axis=-1, keepdims=True)
        cent = emb - mean
        var = jnp.mean(cent * cent, axis=-1, keepdims=True)
        inv = lax.rsqrt(var + float(ln_eps))
        o_ref[...] = cent * inv * gam_ref[...] + bet_ref[...]

    in_specs = [
        pl.BlockSpec((BT, H // 2 if packed else H), lambda i: (i, 0)),
        pl.BlockSpec((1, 1, BT), lambda i: (i + base, 0, 0)),
        pl.BlockSpec((1, 1, BT), lambda i: (i + base, 0, 0)),
        pl.BlockSpec((_COMB_ROWS, H), lambda i: (0, 0)),
        pl.BlockSpec((S, H), lambda i: (0, 0)),
        pl.BlockSpec((1, H), lambda i: (0, 0)),
        pl.BlockSpec((1, H), lambda i: (0, 0)),
    ]
    args = [gathered_c, age_r, tt_r, comb, pos_emb, gamma2, beta2]
    aliases = {}
    if acc is not None:
        in_specs = [pl.BlockSpec(memory_space=pl.ANY)] + in_specs
        args = [acc] + args
        aliases = {0: 0}

    return pl.pallas_call(
        body,
        grid=(NBc,),
        in_specs=in_specs,
        out_specs=pl.BlockSpec((BT, H), lambda i: (i + base, 0)),
        out_shape=jax.ShapeDtypeStruct((T, H), jnp.float32),
        input_output_aliases=aliases,
        compiler_params=pltpu.CompilerParams(
            dimension_semantics=("arbitrary",)),
    )(*args)


def kernel(input_ids, age_ids, token_type_ids, word_emb, token_type_emb,
           age_emb, pos_emb, ln_gamma, ln_beta):
    B, S = input_ids.shape
    H = word_emb.shape[1]
    T = B * S
    C = _NUM_CHUNKS
    Tc = T // C
    BT = _TC_BLOCK_TOKENS
    NBc = Tc // BT
    NB = T // BT
    AGES = age_emb.shape[0]

    word16 = word_emb.astype(jnp.bfloat16)
    lo = lax.bitcast_convert_type(word16[:, :H // 2],
                                  jnp.uint16).astype(jnp.uint32)
    hi = lax.bitcast_convert_type(word16[:, H // 2:],
                                  jnp.uint16).astype(jnp.uint32)
    word_i32 = lax.bitcast_convert_type(lo | (hi << 16), jnp.int32)

    comb = jnp.zeros((_COMB_ROWS, H), jnp.bfloat16)
    comb = comb.at[:AGES].set(age_emb.astype(jnp.bfloat16))
    comb = comb.at[AGES:AGES + token_type_emb.shape[0]].set(
        token_type_emb.astype(jnp.bfloat16))

    ids2 = input_ids.reshape(1, T)
    age_r = age_ids.reshape(NB, 1, BT)
    tt_r = token_type_ids.reshape(NB, 1, BT)
    gamma2 = ln_gamma.reshape(1, H)
    beta2 = ln_beta.reshape(1, H)

    F = _F32_CHUNKS
    gathered = [_sc_gather_chunk(word_emb, ids2, c, Tc) for c in range(F)]
    gathered += [_sc_gather_chunk(word_i32, ids2, c, Tc)
                 for c in range(F, C)]
    acc = None
    for c in range(C):
        acc = _tc_fuse_chunk(acc, gathered[c], age_r, tt_r, comb,
                             pos_emb, gamma2, beta2, c, T, 1e-12,
                             packed=c >= F)
    return acc.reshape(B, S, H)


# speedup vs baseline: 1.1007x; 1.1007x over previous
import functools

import jax
import jax.numpy as jnp
from jax import lax
from jax.experimental import pallas as pl
from jax.experimental.pallas import tpu as pltpu
from jax.experimental.pallas import tpu_sc as plsc

_GATHER_WINDOW = 128
_TC_BLOCK_TOKENS = 2048
_COMB_ROWS = 128
_NUM_CHUNKS = 8
_F32_CHUNKS = 1
_PACK_BLOCK_ROWS = 2000


def _pack_table(word_emb):
    V, H = word_emb.shape
    R = _PACK_BLOCK_ROWS

    def body(w_ref, o_ref):
        w = lax.bitcast_convert_type(w_ref[...], jnp.int32)
        a = w[:, :H // 2]
        b = w[:, H // 2:]

        def rne(x):
            return (x + 0x7FFF + ((x >> 16) & 1)) >> 16

        lo = jnp.bitwise_and(rne(a), jnp.int32(0xFFFF))
        hi = rne(b) << 16
        o_ref[...] = lo | hi

    return pl.pallas_call(
        body,
        grid=(V // R,),
        in_specs=[pl.BlockSpec((R, H), lambda i: (i, 0))],
        out_specs=pl.BlockSpec((R, H // 2), lambda i: (i, 0)),
        out_shape=jax.ShapeDtypeStruct((V, H // 2), jnp.int32),
        compiler_params=pltpu.CompilerParams(
            dimension_semantics=("arbitrary",)),
    )(word_emb)


def _sc_gather_chunk(table, ids2, chunk, Tc):
    H = table.shape[1]
    W = _GATHER_WINDOW
    steps = Tc // W
    off = chunk * steps
    mesh = plsc.VectorSubcoreMesh(core_axis_name="c", subcore_axis_name="s")

    @functools.partial(
        pl.kernel,
        out_type=jax.ShapeDtypeStruct((Tc, H), table.dtype),
        mesh=mesh,
    )
    def gather_kernel(x_hbm, i_hbm, o_hbm):
        def body(i_vmem, o_vmem):
            pltpu.sync_copy(x_hbm.at[i_vmem.at[0]], o_vmem)

        pltpu.emit_pipeline(
            body,
            grid=(steps,),
            in_specs=[
                pl.BlockSpec((1, W), index_map=lambda i: (0, i + off))
            ],
            out_specs=[
                pl.BlockSpec((W, H), index_map=lambda i: (i, 0))
            ],
            core_axis_name=("c", "s"),
            dimension_semantics=(pltpu.PARALLEL,),
        )(i_hbm, o_hbm)

    return gather_kernel(table, ids2)


def _tc_fuse_chunk(acc, gathered_c, age_r, tt_r, comb, pos_emb, gamma2, beta2,
                   chunk, T, ln_eps, packed):
    Tc = gathered_c.shape[0]
    H = 2 * gathered_c.shape[1] if packed else gathered_c.shape[1]
    S = pos_emb.shape[0]
    BT = _TC_BLOCK_TOKENS
    NBc = Tc // BT
    KB = BT // S
    AGES = 110
    base = chunk * NBc

    def body(*refs):
        if acc is None:
            g_ref, age_ref, tt_ref, comb_ref, pos_ref, gam_ref, bet_ref, \
                o_ref = refs
        else:
            _, g_ref, age_ref, tt_ref, comb_ref, pos_ref, gam_ref, bet_ref, \
                o_ref = refs
        if packed:
            g32 = g_ref[...]
            lo_f = lax.bitcast_convert_type(g32 << 16, jnp.float32)
            hi_f = lax.bitcast_convert_type(
                jnp.bitwise_and(g32, jnp.int32(-65536)), jnp.float32)
            g = jnp.concatenate([lo_f, hi_f], axis=1)
        else:
            g = g_ref[...]
        age = age_ref[0, 0, :][:, None]
        tt = tt_ref[0, 0, :][:, None]

        col = lax.broadcasted_iota(jnp.int32, (1, _COMB_ROWS), 1)
        oh = ((age == col).astype(jnp.bfloat16)
              + (tt + AGES == col).astype(jnp.bfloat16))
        small_v = jnp.dot(oh, comb_ref[...],
                          preferred_element_type=jnp.float32)

        pos = jnp.broadcast_to(pos_ref[...][None], (KB, S, H)).reshape(BT, H)

        emb = g + small_v + pos
        mean = jnp.mean(emb, axis=-1, keepdims=True)
        cent = emb - mean
        var = jnp.mean(cent * cent, axis=-1, keepdims=True)
        inv = lax.rsqrt(var + float(ln_eps))
        o_ref[...] = cent * inv * gam_ref[...] + bet_ref[...]

    in_specs = [
        pl.BlockSpec((BT, H // 2 if packed else H), lambda i: (i, 0)),
        pl.BlockSpec((1, 1, BT), lambda i: (i + base, 0, 0)),
        pl.BlockSpec((1, 1, BT), lambda i: (i + base, 0, 0)),
        pl.BlockSpec((_COMB_ROWS, H), lambda i: (0, 0)),
        pl.BlockSpec((S, H), lambda i: (0, 0)),
        pl.BlockSpec((1, H), lambda i: (0, 0)),
        pl.BlockSpec((1, H), lambda i: (0, 0)),
    ]
    args = [gathered_c, age_r, tt_r, comb, pos_emb, gamma2, beta2]
    aliases = {}
    if acc is not None:
        in_specs = [pl.BlockSpec(memory_space=pl.ANY)] + in_specs
        args = [acc] + args
        aliases = {0: 0}

    return pl.pallas_call(
        body,
        grid=(NBc,),
        in_specs=in_specs,
        out_specs=pl.BlockSpec((BT, H), lambda i: (i + base, 0)),
        out_shape=jax.ShapeDtypeStruct((T, H), jnp.float32),
        input_output_aliases=aliases,
        compiler_params=pltpu.CompilerParams(
            dimension_semantics=("arbitrary",)),
    )(*args)


def kernel(input_ids, age_ids, token_type_ids, word_emb, token_type_emb,
           age_emb, pos_emb, ln_gamma, ln_beta):
    B, S = input_ids.shape
    H = word_emb.shape[1]
    T = B * S
    C = _NUM_CHUNKS
    Tc = T // C
    BT = _TC_BLOCK_TOKENS
    NBc = Tc // BT
    NB = T // BT
    AGES = age_emb.shape[0]

    word_i32 = _pack_table(word_emb)

    comb = jnp.zeros((_COMB_ROWS, H), jnp.bfloat16)
    comb = comb.at[:AGES].set(age_emb.astype(jnp.bfloat16))
    comb = comb.at[AGES:AGES + token_type_emb.shape[0]].set(
        token_type_emb.astype(jnp.bfloat16))

    ids2 = input_ids.reshape(1, T)
    age_r = age_ids.reshape(NB, 1, BT)
    tt_r = token_type_ids.reshape(NB, 1, BT)
    gamma2 = ln_gamma.reshape(1, H)
    beta2 = ln_beta.reshape(1, H)

    F = _F32_CHUNKS
    gathered = [_sc_gather_chunk(word_emb, ids2, c, Tc) for c in range(F)]
    gathered += [_sc_gather_chunk(word_i32, ids2, c, Tc)
                 for c in range(F, C)]
    acc = None
    for c in range(C):
        acc = _tc_fuse_chunk(acc, gathered[c], age_r, tt_r, comb,
                             pos_emb, gamma2, beta2, c, T, 1e-12,
                             packed=c >= F)
    return acc.reshape(B, S, H)


# speedup vs baseline: 1.1488x; 1.0437x over previous
import functools

import jax
import jax.numpy as jnp
from jax import lax
from jax.experimental import pallas as pl
from jax.experimental.pallas import tpu as pltpu
from jax.experimental.pallas import tpu_sc as plsc

_GATHER_WINDOW = 128
_TC_BLOCK_TOKENS = 2048
_COMB_ROWS = 128
_NUM_CHUNKS = 8
_F32_CHUNKS = 0
_PACK_BLOCK_ROWS = 2000


def _pack_table(word_emb):
    V, H = word_emb.shape
    R = _PACK_BLOCK_ROWS

    def body(w_ref, o_ref):
        w = lax.bitcast_convert_type(w_ref[...], jnp.int32)
        a = w[:, :H // 2]
        b = w[:, H // 2:]

        def rne(x):
            return (x + 0x7FFF + ((x >> 16) & 1)) >> 16

        lo = jnp.bitwise_and(rne(a), jnp.int32(0xFFFF))
        hi = rne(b) << 16
        o_ref[...] = lo | hi

    return pl.pallas_call(
        body,
        grid=(V // R,),
        in_specs=[pl.BlockSpec((R, H), lambda i: (i, 0))],
        out_specs=pl.BlockSpec((R, H // 2), lambda i: (i, 0)),
        out_shape=jax.ShapeDtypeStruct((V, H // 2), jnp.int32),
        compiler_params=pltpu.CompilerParams(
            dimension_semantics=("arbitrary",)),
    )(word_emb)


def _sc_gather_chunk(table, ids, chunk, Tc):
    H = table.shape[1]
    W = _GATHER_WINDOW
    S = ids.shape[1]
    wpr = S // W
    steps = Tc // W
    off = chunk * steps
    mesh = plsc.VectorSubcoreMesh(core_axis_name="c", subcore_axis_name="s")

    @functools.partial(
        pl.kernel,
        out_type=jax.ShapeDtypeStruct((Tc, H), table.dtype),
        mesh=mesh,
    )
    def gather_kernel(x_hbm, i_hbm, o_hbm):
        def body(i_vmem, o_vmem):
            pltpu.sync_copy(x_hbm.at[i_vmem.at[0]], o_vmem)

        pltpu.emit_pipeline(
            body,
            grid=(steps,),
            in_specs=[
                pl.BlockSpec(
                    (1, W),
                    index_map=lambda i: ((i + off) // wpr, (i + off) % wpr))
            ],
            out_specs=[
                pl.BlockSpec((W, H), index_map=lambda i: (i, 0))
            ],
            core_axis_name=("c", "s"),
            dimension_semantics=(pltpu.PARALLEL,),
        )(i_hbm, o_hbm)

    return gather_kernel(table, ids)


def _tc_fuse_chunk(acc, gathered_c, age_r, tt_r, comb, pos_emb, gamma2, beta2,
                   chunk, T, ln_eps, packed):
    Tc = gathered_c.shape[0]
    H = 2 * gathered_c.shape[1] if packed else gathered_c.shape[1]
    S = pos_emb.shape[0]
    BT = _TC_BLOCK_TOKENS
    NBc = Tc // BT
    KB = BT // S
    AGES = 110
    base = chunk * NBc

    def body(*refs):
        if acc is None:
            g_ref, age_ref, tt_ref, comb_ref, pos_ref, gam_ref, bet_ref, \
                o_ref = refs
        else:
            _, g_ref, age_ref, tt_ref, comb_ref, pos_ref, gam_ref, bet_ref, \
                o_ref = refs
        if packed:
            g32 = g_ref[...]
            lo_f = lax.bitcast_convert_type(g32 << 16, jnp.float32)
            hi_f = lax.bitcast_convert_type(
                jnp.bitwise_and(g32, jnp.int32(-65536)), jnp.float32)
            g = jnp.concatenate([lo_f, hi_f], axis=1)
        else:
            g = g_ref[...]
        age = age_ref[0, 0, :][:, None]
        tt = tt_ref[0, 0, :][:, None]

        col = lax.broadcasted_iota(jnp.int32, (1, _COMB_ROWS), 1)
        oh = ((age == col).astype(jnp.bfloat16)
              + (tt + AGES == col).astype(jnp.bfloat16))
        small_v = jnp.dot(oh, comb_ref[...],
                          preferred_element_type=jnp.float32)

        pos = jnp.broadcast_to(pos_ref[...][None], (KB, S, H)).reshape(BT, H)

        emb = g + small_v + pos
        mean = jnp.mean(emb, axis=-1, keepdims=True)
        cent = emb - mean
        var = jnp.mean(cent * cent, axis=-1, keepdims=True)
        inv = lax.rsqrt(var + float(ln_eps))
        o_ref[...] = cent * inv * gam_ref[...] + bet_ref[...]

    in_specs = [
        pl.BlockSpec((BT, H // 2 if packed else H), lambda i: (i, 0)),
        pl.BlockSpec((1, 1, BT), lambda i: (i + base, 0, 0)),
        pl.BlockSpec((1, 1, BT), lambda i: (i + base, 0, 0)),
        pl.BlockSpec((_COMB_ROWS, H), lambda i: (0, 0)),
        pl.BlockSpec((S, H), lambda i: (0, 0)),
        pl.BlockSpec((1, H), lambda i: (0, 0)),
        pl.BlockSpec((1, H), lambda i: (0, 0)),
    ]
    args = [gathered_c, age_r, tt_r, comb, pos_emb, gamma2, beta2]
    aliases = {}
    if acc is not None:
        in_specs = [pl.BlockSpec(memory_space=pl.ANY)] + in_specs
        args = [acc] + args
        aliases = {0: 0}

    return pl.pallas_call(
        body,
        grid=(NBc,),
        in_specs=in_specs,
        out_specs=pl.BlockSpec((BT, H), lambda i: (i + base, 0)),
        out_shape=jax.ShapeDtypeStruct((T, H), jnp.float32),
        input_output_aliases=aliases,
        compiler_params=pltpu.CompilerParams(
            dimension_semantics=("arbitrary",)),
    )(*args)


def kernel(input_ids, age_ids, token_type_ids, word_emb, token_type_emb,
           age_emb, pos_emb, ln_gamma, ln_beta):
    B, S = input_ids.shape
    H = word_emb.shape[1]
    T = B * S
    C = _NUM_CHUNKS
    Tc = T // C
    BT = _TC_BLOCK_TOKENS
    NBc = Tc // BT
    NB = T // BT
    AGES = age_emb.shape[0]

    word_i32 = _pack_table(word_emb)

    comb = jnp.zeros((_COMB_ROWS, H), jnp.bfloat16)
    comb = comb.at[:AGES].set(age_emb.astype(jnp.bfloat16))
    comb = comb.at[AGES:AGES + token_type_emb.shape[0]].set(
        token_type_emb.astype(jnp.bfloat16))

    age_r = age_ids.reshape(NB, 1, BT)
    tt_r = token_type_ids.reshape(NB, 1, BT)
    gamma2 = ln_gamma.reshape(1, H)
    beta2 = ln_beta.reshape(1, H)

    F = _F32_CHUNKS
    gathered = [_sc_gather_chunk(word_emb, input_ids, c, Tc)
                for c in range(F)]
    gathered += [_sc_gather_chunk(word_i32, input_ids, c, Tc)
                 for c in range(F, C)]
    acc = None
    for c in range(C):
        acc = _tc_fuse_chunk(acc, gathered[c], age_r, tt_r, comb,
                             pos_emb, gamma2, beta2, c, T, 1e-12,
                             packed=c >= F)
    return acc.reshape(B, S, H)


# speedup vs baseline: 1.1905x; 1.0363x over previous
import functools

import jax
import jax.numpy as jnp
from jax import lax
from jax.experimental import pallas as pl
from jax.experimental.pallas import tpu as pltpu
from jax.experimental.pallas import tpu_sc as plsc

_GATHER_WINDOW = 128
_TC_BLOCK_TOKENS = 2048
_COMB_ROWS = 128
_NUM_CHUNKS = 8
_F32_CHUNKS = 0
_PACK_BLOCK_ROWS = 4000


def _pack_table(word_emb):
    V, H = word_emb.shape
    R = _PACK_BLOCK_ROWS

    def body(w_ref, o_ref):
        w = lax.bitcast_convert_type(w_ref[...], jnp.int32)
        a = w[:, :H // 2]
        b = w[:, H // 2:]

        def rne(x):
            return (x + 0x7FFF + ((x >> 16) & 1)) >> 16

        lo = jnp.bitwise_and(rne(a), jnp.int32(0xFFFF))
        hi = rne(b) << 16
        o_ref[...] = lo | hi

    return pl.pallas_call(
        body,
        grid=(V // R,),
        in_specs=[pl.BlockSpec((R, H), lambda i: (i, 0))],
        out_specs=pl.BlockSpec((R, H // 2), lambda i: (i, 0)),
        out_shape=jax.ShapeDtypeStruct((V, H // 2), jnp.int32),
        compiler_params=pltpu.CompilerParams(
            dimension_semantics=("arbitrary",)),
    )(word_emb)


def _sc_gather_chunk(table, ids, block_off, nb):
    H = table.shape[1]
    W = _GATHER_WINDOW
    S = ids.shape[1]
    Tc = nb * _TC_BLOCK_TOKENS
    wpr = S // W
    steps = Tc // W
    off = block_off * (_TC_BLOCK_TOKENS // W)
    mesh = plsc.VectorSubcoreMesh(core_axis_name="c", subcore_axis_name="s")

    @functools.partial(
        pl.kernel,
        out_type=jax.ShapeDtypeStruct((Tc, H), table.dtype),
        mesh=mesh,
    )
    def gather_kernel(x_hbm, i_hbm, o_hbm):
        def body(i_vmem, o_vmem):
            pltpu.sync_copy(x_hbm.at[i_vmem.at[0]], o_vmem)

        pltpu.emit_pipeline(
            body,
            grid=(steps,),
            in_specs=[
                pl.BlockSpec(
                    (1, W),
                    index_map=lambda i: ((i + off) // wpr, (i + off) % wpr))
            ],
            out_specs=[
                pl.BlockSpec((W, H), index_map=lambda i: (i, 0))
            ],
            core_axis_name=("c", "s"),
            dimension_semantics=(pltpu.PARALLEL,),
        )(i_hbm, o_hbm)

    return gather_kernel(table, ids)


def _tc_fuse_chunk(acc, gathered_c, age_r, tt_r, comb, pos_emb, gamma2, beta2,
                   block_off, T, ln_eps, packed):
    Tc = gathered_c.shape[0]
    H = 2 * gathered_c.shape[1] if packed else gathered_c.shape[1]
    S = pos_emb.shape[0]
    BT = _TC_BLOCK_TOKENS
    NBc = Tc // BT
    KB = BT // S
    AGES = 110
    base = block_off

    def body(*refs):
        if acc is None:
            g_ref, age_ref, tt_ref, comb_ref, pos_ref, gam_ref, bet_ref, \
                o_ref = refs
        else:
            _, g_ref, age_ref, tt_ref, comb_ref, pos_ref, gam_ref, bet_ref, \
                o_ref = refs
        if packed:
            g32 = g_ref[...]
            lo_f = lax.bitcast_convert_type(g32 << 16, jnp.float32)
            hi_f = lax.bitcast_convert_type(
                jnp.bitwise_and(g32, jnp.int32(-65536)), jnp.float32)
            g = jnp.concatenate([lo_f, hi_f], axis=1)
        else:
            g = g_ref[...]
        age = age_ref[0, 0, :][:, None]
        tt = tt_ref[0, 0, :][:, None]

        col = lax.broadcasted_iota(jnp.int32, (1, _COMB_ROWS), 1)
        oh = ((age == col).astype(jnp.bfloat16)
              + (tt + AGES == col).astype(jnp.bfloat16))
        small_v = jnp.dot(oh, comb_ref[...],
                          preferred_element_type=jnp.float32)

        pos = jnp.broadcast_to(pos_ref[...][None], (KB, S, H)).reshape(BT, H)

        emb = g + small_v + pos
        s1 = jnp.sum(emb, axis=-1, keepdims=True)
        s2 = jnp.sum(emb * emb, axis=-1, keepdims=True)
        mean = s1 * (1.0 / H)
        var = s2 * (1.0 / H) - mean * mean
        inv = lax.rsqrt(var + float(ln_eps))
        o_ref[...] = (emb - mean) * inv * gam_ref[...] + bet_ref[...]

    in_specs = [
        pl.BlockSpec((BT, H // 2 if packed else H), lambda i: (i, 0)),
        pl.BlockSpec((1, 1, BT), lambda i: (i + base, 0, 0)),
        pl.BlockSpec((1, 1, BT), lambda i: (i + base, 0, 0)),
        pl.BlockSpec((_COMB_ROWS, H), lambda i: (0, 0)),
        pl.BlockSpec((S, H), lambda i: (0, 0)),
        pl.BlockSpec((1, H), lambda i: (0, 0)),
        pl.BlockSpec((1, H), lambda i: (0, 0)),
    ]
    args = [gathered_c, age_r, tt_r, comb, pos_emb, gamma2, beta2]
    aliases = {}
    if acc is not None:
        in_specs = [pl.BlockSpec(memory_space=pl.ANY)] + in_specs
        args = [acc] + args
        aliases = {0: 0}

    return pl.pallas_call(
        body,
        grid=(NBc,),
        in_specs=in_specs,
        out_specs=pl.BlockSpec((BT, H), lambda i: (i + base, 0)),
        out_shape=jax.ShapeDtypeStruct((T, H), jnp.float32),
        input_output_aliases=aliases,
        compiler_params=pltpu.CompilerParams(
            dimension_semantics=("arbitrary",)),
    )(*args)


def kernel(input_ids, age_ids, token_type_ids, word_emb, token_type_emb,
           age_emb, pos_emb, ln_gamma, ln_beta):
    B, S = input_ids.shape
    H = word_emb.shape[1]
    T = B * S
    BT = _TC_BLOCK_TOKENS
    NB = T // BT
    AGES = age_emb.shape[0]
    chunk_blocks = ([4, 8, 16, 20, 20, 20, 20, 20] if NB == 128 else [NB])
    assert sum(chunk_blocks) == NB
    offs = [sum(chunk_blocks[:c]) for c in range(len(chunk_blocks))]

    word_i32 = _pack_table(word_emb)

    comb = jnp.zeros((_COMB_ROWS, H), jnp.bfloat16)
    comb = comb.at[:AGES].set(age_emb.astype(jnp.bfloat16))
    comb = comb.at[AGES:AGES + token_type_emb.shape[0]].set(
        token_type_emb.astype(jnp.bfloat16))

    age_r = age_ids.reshape(NB, 1, BT)
    tt_r = token_type_ids.reshape(NB, 1, BT)
    gamma2 = ln_gamma.reshape(1, H)
    beta2 = ln_beta.reshape(1, H)

    gathered = [_sc_gather_chunk(word_i32, input_ids, offs[c], nb)
                for c, nb in enumerate(chunk_blocks)]
    acc = None
    for c, nb in enumerate(chunk_blocks):
        acc = _tc_fuse_chunk(acc, gathered[c], age_r, tt_r, comb,
                             pos_emb, gamma2, beta2, offs[c], T, 1e-12,
                             packed=True)
    return acc.reshape(B, S, H)


# speedup vs baseline: 1.1931x; 1.0022x over previous
import functools

import jax
import jax.numpy as jnp
from jax import lax
from jax.experimental import pallas as pl
from jax.experimental.pallas import tpu as pltpu
from jax.experimental.pallas import tpu_sc as plsc

_GATHER_WINDOW = 128
_TC_BLOCK_TOKENS = 2048
_COMB_ROWS = 128
_NUM_CHUNKS = 8
_F32_CHUNKS = 0
_PACK_BLOCK_ROWS = 4000


def _pack_table(word_emb):
    V, H = word_emb.shape
    R = _PACK_BLOCK_ROWS

    def body(w_ref, o_ref):
        w = lax.bitcast_convert_type(w_ref[...], jnp.int32)
        a = w[:, :H // 2]
        b = w[:, H // 2:]

        def rne(x):
            return (x + 0x7FFF + ((x >> 16) & 1)) >> 16

        lo = jnp.bitwise_and(rne(a), jnp.int32(0xFFFF))
        hi = rne(b) << 16
        o_ref[...] = lo | hi

    return pl.pallas_call(
        body,
        grid=(V // R,),
        in_specs=[pl.BlockSpec((R, H), lambda i: (i, 0))],
        out_specs=pl.BlockSpec((R, H // 2), lambda i: (i, 0)),
        out_shape=jax.ShapeDtypeStruct((V, H // 2), jnp.int32),
        compiler_params=pltpu.CompilerParams(
            dimension_semantics=("arbitrary",)),
    )(word_emb)


def _sc_gather_chunk(table, ids, block_off, nb):
    H = table.shape[1]
    W = _GATHER_WINDOW
    S = ids.shape[1]
    Tc = nb * _TC_BLOCK_TOKENS
    wpr = S // W
    steps = Tc // W
    off = block_off * (_TC_BLOCK_TOKENS // W)
    mesh = plsc.VectorSubcoreMesh(core_axis_name="c", subcore_axis_name="s")

    @functools.partial(
        pl.kernel,
        out_type=jax.ShapeDtypeStruct((Tc, H), table.dtype),
        mesh=mesh,
    )
    def gather_kernel(x_hbm, i_hbm, o_hbm):
        def body(i_vmem, o_vmem):
            pltpu.sync_copy(x_hbm.at[i_vmem.at[0]], o_vmem)

        pltpu.emit_pipeline(
            body,
            grid=(steps,),
            in_specs=[
                pl.BlockSpec(
                    (1, W),
                    index_map=lambda i: ((i + off) // wpr, (i + off) % wpr))
            ],
            out_specs=[
                pl.BlockSpec((W, H), index_map=lambda i: (i, 0))
            ],
            core_axis_name=("c", "s"),
            dimension_semantics=(pltpu.PARALLEL,),
        )(i_hbm, o_hbm)

    return gather_kernel(table, ids)


def _tc_fuse_chunk(acc, gathered_c, age_r, tt_r, comb, pos_emb, gamma2, beta2,
                   block_off, T, ln_eps, packed):
    Tc = gathered_c.shape[0]
    H = 2 * gathered_c.shape[1] if packed else gathered_c.shape[1]
    S = pos_emb.shape[0]
    BT = _TC_BLOCK_TOKENS
    NBc = Tc // BT
    KB = BT // S
    AGES = 110
    base = block_off

    def body(*refs):
        if acc is None:
            g_ref, age_ref, tt_ref, comb_ref, pos_ref, gam_ref, bet_ref, \
                o_ref = refs
        else:
            _, g_ref, age_ref, tt_ref, comb_ref, pos_ref, gam_ref, bet_ref, \
                o_ref = refs
        if packed:
            g32 = g_ref[...]
            lo_f = lax.bitcast_convert_type(g32 << 16, jnp.float32)
            hi_f = lax.bitcast_convert_type(
                jnp.bitwise_and(g32, jnp.int32(-65536)), jnp.float32)
            g = jnp.concatenate([lo_f, hi_f], axis=1)
        else:
            g = g_ref[...]
        age = age_ref[0, 0, :][:, None]
        tt = tt_ref[0, 0, :][:, None]

        col = lax.broadcasted_iota(jnp.int32, (1, _COMB_ROWS), 1)
        oh = ((age == col).astype(jnp.bfloat16)
              + (tt + AGES == col).astype(jnp.bfloat16))
        small_v = jnp.dot(oh, comb_ref[...],
                          preferred_element_type=jnp.float32)

        emb = (g + small_v).reshape(KB, S, H) + pos_ref[...][None]
        s1 = jnp.sum(emb, axis=-1, keepdims=True)
        s2 = jnp.sum(emb * emb, axis=-1, keepdims=True)
        mean = s1 * (1.0 / H)
        var = s2 * (1.0 / H) - mean * mean
        inv = lax.rsqrt(var + float(ln_eps))
        out = (emb - mean) * inv * gam_ref[...][None] + bet_ref[...][None]
        o_ref[...] = out.reshape(BT, H)

    in_specs = [
        pl.BlockSpec((BT, H // 2 if packed else H), lambda i: (i, 0)),
        pl.BlockSpec((1, 1, BT), lambda i: (i + base, 0, 0)),
        pl.BlockSpec((1, 1, BT), lambda i: (i + base, 0, 0)),
        pl.BlockSpec((_COMB_ROWS, H), lambda i: (0, 0)),
        pl.BlockSpec((S, H), lambda i: (0, 0)),
        pl.BlockSpec((1, H), lambda i: (0, 0)),
        pl.BlockSpec((1, H), lambda i: (0, 0)),
    ]
    args = [gathered_c, age_r, tt_r, comb, pos_emb, gamma2, beta2]
    aliases = {}
    if acc is not None:
        in_specs = [pl.BlockSpec(memory_space=pl.ANY)] + in_specs
        args = [acc] + args
        aliases = {0: 0}

    return pl.pallas_call(
        body,
        grid=(NBc,),
        in_specs=in_specs,
        out_specs=pl.BlockSpec((BT, H), lambda i: (i + base, 0)),
        out_shape=jax.ShapeDtypeStruct((T, H), jnp.float32),
        input_output_aliases=aliases,
        compiler_params=pltpu.CompilerParams(
            dimension_semantics=("arbitrary",)),
    )(*args)


def kernel(input_ids, age_ids, token_type_ids, word_emb, token_type_emb,
           age_emb, pos_emb, ln_gamma, ln_beta):
    B, S = input_ids.shape
    H = word_emb.shape[1]
    T = B * S
    BT = _TC_BLOCK_TOKENS
    NB = T // BT
    AGES = age_emb.shape[0]
    chunk_blocks = ([4, 8, 16, 20, 20, 20, 20, 20] if NB == 128 else [NB])
    assert sum(chunk_blocks) == NB
    offs = [sum(chunk_blocks[:c]) for c in range(len(chunk_blocks))]

    word_i32 = _pack_table(word_emb)

    comb = jnp.zeros((_COMB_ROWS, H), jnp.bfloat16)
    comb = comb.at[:AGES].set(age_emb.astype(jnp.bfloat16))
    comb = comb.at[AGES:AGES + token_type_emb.shape[0]].set(
        token_type_emb.astype(jnp.bfloat16))

    age_r = age_ids.reshape(NB, 1, BT)
    tt_r = token_type_ids.reshape(NB, 1, BT)
    gamma2 = ln_gamma.reshape(1, H)
    beta2 = ln_beta.reshape(1, H)

    gathered = [_sc_gather_chunk(word_i32, input_ids, offs[c], nb)
                for c, nb in enumerate(chunk_blocks)]
    acc = None
    for c, nb in enumerate(chunk_blocks):
        acc = _tc_fuse_chunk(acc, gathered[c], age_r, tt_r, comb,
                             pos_emb, gamma2, beta2, offs[c], T, 1e-12,
                             packed=True)
    return acc.reshape(B, S, H)
